# baseline (speedup 1.0000x reference)
import jax
import jax.numpy as jnp
from jax.experimental import pallas as pl
from jax.experimental.pallas import tpu as pltpu

_PAD = 1
_HALF = 512
_S_BLK = 1024
_NR = 32
_NQ = _S_BLK // _NR
_RSHIFT = 5


def _freq(shape, dim):
    scale = jnp.log(10000.0) / (_HALF - 1)
    j = jax.lax.broadcasted_iota(jnp.int32, shape, dim).astype(jnp.float32)
    return jnp.exp(j * -scale)


def _prep_body(tok_ref, rtab_ref, qtab_ref, qid_ref, rid_ref, carr_ref):
    f_r = _freq((_NR, _HALF), 1)
    r1 = jax.lax.broadcasted_iota(jnp.int32, (_NR, _HALF), 0)
    arg_r = (r1 + 1).astype(jnp.float32) * f_r
    rtab_ref[:, :_HALF] = jnp.sin(arg_r)
    rtab_ref[:, _HALF:] = jnp.cos(arg_r)
    f_q = _freq((_NQ, _HALF), 1)
    q1 = jax.lax.broadcasted_iota(jnp.int32, (_NQ, _HALF), 0)
    arg_q = (q1 * _NR).astype(jnp.float32) * f_q
    qtab_ref[:, :_HALF] = jnp.sin(arg_q)
    qtab_ref[:, _HALF:] = jnp.cos(arg_q)

    tok = tok_ref[...]
    bsz, seq = tok.shape
    mask = tok != _PAD
    g = mask.astype(jnp.int32)
    k = 1
    while k < seq:
        shifted = jnp.concatenate(
            [jnp.zeros((bsz, k), jnp.int32), g[:, :-k]], axis=1)
        g = g + shifted
        k *= 2

    nblk = seq // _S_BLK
    carr_cols = [jnp.zeros((bsz, 1), jnp.int32)]
    for s in range(1, nblk):
        carr_cols.append(g[:, s * _S_BLK - 1:s * _S_BLK])
    carr = jnp.concatenate(carr_cols, axis=1)
    carr_ref[...] = carr.astype(jnp.float32)

    c_bcast = jnp.concatenate(
        [jnp.broadcast_to(carr[:, s:s + 1], (bsz, _S_BLK))
         for s in range(nblk)], axis=1)
    lm1 = g - c_bcast - 1
    qid_ref[...] = jax.lax.shift_right_logical(lm1, jnp.int32(_RSHIFT))
    rid_ref[...] = jnp.where(
        mask, jax.lax.bitwise_and(lm1, jnp.int32(_NR - 1)), -1)


def _emb_body(qid_ref, rid_ref, carr_ref, rtab_ref, qtab_ref, out_ref):
    c = carr_ref[0, 0, 0, 0]
    f_row = _freq((1, _HALF), 1)
    arg_c = (c + jnp.float32(_PAD)) * f_row
    sin_c = jnp.sin(arg_c)
    cos_c = jnp.cos(arg_c)
    s64 = qtab_ref[:, :_HALF]
    c64 = qtab_ref[:, _HALF:]
    aq = jnp.concatenate(
        [sin_c * c64 + cos_c * s64, cos_c * c64 - sin_c * s64], axis=1
    )

    q_id = qid_ref[0, 0]
    r_id = rid_ref[0, 0]
    row_q = jax.lax.broadcasted_iota(jnp.int32, (_NQ, _S_BLK), 0)
    row_r = jax.lax.broadcasted_iota(jnp.int32, (_NR, _S_BLK), 0)
    oh_qt = (q_id == row_q).astype(jnp.bfloat16)
    oh_rt = (r_id == row_r).astype(jnp.bfloat16)

    dn = (((0,), (0,)), ((), ()))
    qr = jax.lax.dot_general(oh_qt, aq.astype(jnp.bfloat16), dn,
                             preferred_element_type=jnp.float32)
    rr = jax.lax.dot_general(oh_rt, rtab_ref[:, :].astype(jnp.bfloat16), dn,
                             preferred_element_type=jnp.float32)
    qs, qc = qr[:, :_HALF], qr[:, _HALF:]
    rs, rc = rr[:, :_HALF], rr[:, _HALF:]
    out_ref[0, :, :_HALF] = qs * rc + qc * rs
    out_ref[0, :, _HALF:] = qc * rc - qs * rs


@jax.jit
def kernel(inputs, weight):
    del weight
    bsz, seq_len = inputs.shape
    nblk = seq_len // _S_BLK
    rtab, qtab, qid, rid, carrf = pl.pallas_call(
        _prep_body,
        out_specs=[
            pl.BlockSpec((_NR, 2 * _HALF), lambda: (0, 0)),
            pl.BlockSpec((_NQ, 2 * _HALF), lambda: (0, 0)),
            pl.BlockSpec((bsz, seq_len), lambda: (0, 0)),
            pl.BlockSpec((bsz, seq_len), lambda: (0, 0)),
            pl.BlockSpec((bsz, nblk), lambda: (0, 0)),
        ],
        out_shape=[
            jax.ShapeDtypeStruct((_NR, 2 * _HALF), jnp.float32),
            jax.ShapeDtypeStruct((_NQ, 2 * _HALF), jnp.float32),
            jax.ShapeDtypeStruct((bsz, seq_len), jnp.int32),
            jax.ShapeDtypeStruct((bsz, seq_len), jnp.int32),
            jax.ShapeDtypeStruct((bsz, nblk), jnp.float32),
        ],
    )(inputs)
    qid4 = qid.reshape(bsz, nblk, 1, _S_BLK)
    rid4 = rid.reshape(bsz, nblk, 1, _S_BLK)
    carr4 = carrf.reshape(bsz, nblk, 1, 1)
    out = pl.pallas_call(
        _emb_body,
        grid=(bsz, nblk),
        in_specs=[
            pl.BlockSpec((1, 1, 1, _S_BLK), lambda b, s: (b, s, 0, 0)),
            pl.BlockSpec((1, 1, 1, _S_BLK), lambda b, s: (b, s, 0, 0)),
            pl.BlockSpec((1, 1, 1, 1), lambda b, s: (b, s, 0, 0)),
            pl.BlockSpec((_NR, 2 * _HALF), lambda b, s: (0, 0)),
            pl.BlockSpec((_NQ, 2 * _HALF), lambda b, s: (0, 0)),
        ],
        out_specs=pl.BlockSpec((1, _S_BLK, 2 * _HALF), lambda b, s: (b, s, 0)),
        out_shape=jax.ShapeDtypeStruct((bsz, seq_len, 2 * _HALF), jnp.float32),
        compiler_params=pltpu.CompilerParams(
            dimension_semantics=("parallel", "parallel"),
        ),
    )(qid4, rid4, carr4, rtab, qtab)
    return jax.lax.stop_gradient(out)


# speedup vs baseline: 1.0842x; 1.0842x over previous
import jax
import jax.numpy as jnp
from jax.experimental import pallas as pl
from jax.experimental.pallas import tpu as pltpu

_PAD = 1
_HALF = 512
_S_BLK = 2048
_NR = 32
_NQ = _S_BLK // _NR
_RSHIFT = 5


def _freq(shape, dim):
    scale = jnp.log(10000.0) / (_HALF - 1)
    j = jax.lax.broadcasted_iota(jnp.int32, shape, dim).astype(jnp.float32)
    return jnp.exp(j * -scale)


def _prep_body(tok_ref, rtab_ref, qtab_ref, qid_ref, rid_ref, carr_ref):
    f_r = _freq((_NR, _HALF), 1)
    r1 = jax.lax.broadcasted_iota(jnp.int32, (_NR, _HALF), 0)
    arg_r = (r1 + 1).astype(jnp.float32) * f_r
    rtab_ref[:, :_HALF] = jnp.sin(arg_r)
    rtab_ref[:, _HALF:] = jnp.cos(arg_r)
    f_q = _freq((_NQ, _HALF), 1)
    q1 = jax.lax.broadcasted_iota(jnp.int32, (_NQ, _HALF), 0)
    arg_q = (q1 * _NR).astype(jnp.float32) * f_q
    qtab_ref[:, :_HALF] = jnp.sin(arg_q)
    qtab_ref[:, _HALF:] = jnp.cos(arg_q)

    tok = tok_ref[...]
    bsz, seq = tok.shape
    mask = tok != _PAD
    g = mask.astype(jnp.int32)
    k = 1
    while k < seq:
        shifted = jnp.concatenate(
            [jnp.zeros((bsz, k), jnp.int32), g[:, :-k]], axis=1)
        g = g + shifted
        k *= 2

    nblk = seq // _S_BLK
    carr_cols = [jnp.zeros((bsz, 1), jnp.int32)]
    for s in range(1, nblk):
        carr_cols.append(g[:, s * _S_BLK - 1:s * _S_BLK])
    carr = jnp.concatenate(carr_cols, axis=1)
    carr_ref[...] = carr.astype(jnp.float32)

    c_bcast = jnp.concatenate(
        [jnp.broadcast_to(carr[:, s:s + 1], (bsz, _S_BLK))
         for s in range(nblk)], axis=1)
    lm1 = g - c_bcast - 1
    qid_ref[...] = jax.lax.shift_right_logical(lm1, jnp.int32(_RSHIFT))
    rid_ref[...] = jnp.where(
        mask, jax.lax.bitwise_and(lm1, jnp.int32(_NR - 1)), -1)


def _emb_body(qid_ref, rid_ref, carr_ref, rtab_ref, qtab_ref, out_ref):
    c = carr_ref[0, 0, 0, 0]
    f_row = _freq((1, _HALF), 1)
    arg_c = (c + jnp.float32(_PAD)) * f_row
    sin_c = jnp.sin(arg_c)
    cos_c = jnp.cos(arg_c)
    s64 = qtab_ref[:, :_HALF]
    c64 = qtab_ref[:, _HALF:]
    aq = jnp.concatenate(
        [sin_c * c64 + cos_c * s64, cos_c * c64 - sin_c * s64], axis=1
    )

    q_id = qid_ref[0, 0]
    r_id = rid_ref[0, 0]
    row_q = jax.lax.broadcasted_iota(jnp.int32, (_NQ, _S_BLK), 0)
    row_r = jax.lax.broadcasted_iota(jnp.int32, (_NR, _S_BLK), 0)
    oh_qt = (q_id == row_q).astype(jnp.bfloat16)
    oh_rt = (r_id == row_r).astype(jnp.bfloat16)

    dn = (((0,), (0,)), ((), ()))
    qr = jax.lax.dot_general(oh_qt, aq.astype(jnp.bfloat16), dn,
                             preferred_element_type=jnp.float32)
    rr = jax.lax.dot_general(oh_rt, rtab_ref[:, :].astype(jnp.bfloat16), dn,
                             preferred_element_type=jnp.float32)
    qs, qc = qr[:, :_HALF], qr[:, _HALF:]
    rs, rc = rr[:, :_HALF], rr[:, _HALF:]
    out_ref[0, :, :_HALF] = qs * rc + qc * rs
    out_ref[0, :, _HALF:] = qc * rc - qs * rs


@jax.jit
def kernel(inputs, weight):
    del weight
    bsz, seq_len = inputs.shape
    nblk = seq_len // _S_BLK
    rtab, qtab, qid, rid, carrf = pl.pallas_call(
        _prep_body,
        out_specs=[
            pl.BlockSpec((_NR, 2 * _HALF), lambda: (0, 0)),
            pl.BlockSpec((_NQ, 2 * _HALF), lambda: (0, 0)),
            pl.BlockSpec((bsz, seq_len), lambda: (0, 0)),
            pl.BlockSpec((bsz, seq_len), lambda: (0, 0)),
            pl.BlockSpec((bsz, nblk), lambda: (0, 0)),
        ],
        out_shape=[
            jax.ShapeDtypeStruct((_NR, 2 * _HALF), jnp.float32),
            jax.ShapeDtypeStruct((_NQ, 2 * _HALF), jnp.float32),
            jax.ShapeDtypeStruct((bsz, seq_len), jnp.int32),
            jax.ShapeDtypeStruct((bsz, seq_len), jnp.int32),
            jax.ShapeDtypeStruct((bsz, nblk), jnp.float32),
        ],
    )(inputs)
    qid4 = qid.reshape(bsz, nblk, 1, _S_BLK)
    rid4 = rid.reshape(bsz, nblk, 1, _S_BLK)
    carr4 = carrf.reshape(bsz, nblk, 1, 1)
    out = pl.pallas_call(
        _emb_body,
        grid=(bsz, nblk),
        in_specs=[
            pl.BlockSpec((1, 1, 1, _S_BLK), lambda b, s: (b, s, 0, 0)),
            pl.BlockSpec((1, 1, 1, _S_BLK), lambda b, s: (b, s, 0, 0)),
            pl.BlockSpec((1, 1, 1, 1), lambda b, s: (b, s, 0, 0)),
            pl.BlockSpec((_NR, 2 * _HALF), lambda b, s: (0, 0)),
            pl.BlockSpec((_NQ, 2 * _HALF), lambda b, s: (0, 0)),
        ],
        out_specs=pl.BlockSpec((1, _S_BLK, 2 * _HALF), lambda b, s: (b, s, 0)),
        out_shape=jax.ShapeDtypeStruct((bsz, seq_len, 2 * _HALF), jnp.float32),
        compiler_params=pltpu.CompilerParams(
            dimension_semantics=("parallel", "parallel"),
        ),
    )(qid4, rid4, carr4, rtab, qtab)
    return jax.lax.stop_gradient(out)
